# baseline (speedup 1.0000x reference)
import functools

import jax
import jax.numpy as jnp
from jax import lax
from jax.experimental import pallas as pl
from jax.experimental.pallas import tpu as pltpu

N = 10000
E = 320000
F = 128
H = 256
RB = 2000
GRID = N // RB


def _gelu(v):
    return 0.5 * v * (1.0 + lax.erf(v * (2.0 ** -0.5)))


def _dinv_block(dinv_ref):
    i = pl.program_id(0)
    return dinv_ref[pl.ds(i * RB, RB), :]


def _tc0_body(x_ref, degp_ref, dinv_ref, xp_ref):
    deg = jnp.sum(degp_ref[...], axis=0) + 1.0
    dinv_ref[...] = lax.rsqrt(deg)[:, None]
    xp_ref[...] = x_ref[...] * _dinv_block(dinv_ref)


def _tca_body(s1p_ref, xp_ref, dinv_ref, w1_ref, b1_ref, w2_ref, p2_ref):
    dinv = _dinv_block(dinv_ref)
    g = dinv * (s1p_ref[0] + s1p_ref[1] + xp_ref[...])
    z1 = jnp.dot(g, w1_ref[...], preferred_element_type=jnp.float32) + b1_ref[...]
    a1 = _gelu(z1)
    p2 = jnp.dot(a1, w2_ref[...], preferred_element_type=jnp.float32) * dinv
    p2_ref[0] = p2[:, :128]
    p2_ref[1] = p2[:, 128:]


def _tcb_body(scat_ref, pcat_ref, dinv_ref, b_ref, w_ref, pn_ref):
    dinv = _dinv_block(dinv_ref)
    zlo = dinv * (scat_ref[0] + pcat_ref[0])
    zhi = dinv * (scat_ref[1] + pcat_ref[1])
    z = jnp.concatenate([zlo, zhi], axis=1) + b_ref[...]
    a = _gelu(z)
    pn = jnp.dot(a, w_ref[...], preferred_element_type=jnp.float32) * dinv
    pn_ref[0] = pn[:, :128]
    pn_ref[1] = pn[:, 128:]


def _tcc_body(scat_ref, pcat_ref, dinv_ref, b_ref, w4_ref, p4_ref):
    dinv = _dinv_block(dinv_ref)
    zlo = dinv * (scat_ref[0] + pcat_ref[0])
    zhi = dinv * (scat_ref[1] + pcat_ref[1])
    z = jnp.concatenate([zlo, zhi], axis=1) + b_ref[...]
    a = _gelu(z)
    p4_ref[...] = jnp.dot(a, w4_ref[...], preferred_element_type=jnp.float32) * dinv


def _tcd_body(s4p_ref, p4p_ref, dinv_ref, b4_ref, out_ref):
    s4 = jnp.sum(s4p_ref[...], axis=0)
    out_ref[...] = dinv_ref[...] * (s4[:, None] + p4p_ref[...]) + b4_ref[0]


def _row_spec(w):
    return pl.BlockSpec((RB, w), lambda i: (i, 0))


def _half_spec():
    return pl.BlockSpec((2, RB, 128), lambda i: (0, i, 0))


def _full_spec(shape):
    nd = len(shape)
    return pl.BlockSpec(shape, lambda i: (0,) * nd)


def _tc0(x, degp):
    return pl.pallas_call(
        _tc0_body,
        grid=(GRID,),
        in_specs=[_row_spec(F), _full_spec((_NTILE, N))],
        out_specs=[_full_spec((N, 1)), _row_spec(F)],
        out_shape=[jax.ShapeDtypeStruct((N, 1), jnp.float32),
                   jax.ShapeDtypeStruct((N, F), jnp.float32)],
    )(x, degp)


def _tca(s1p, xp, dinv, w1, b1, w2):
    return pl.pallas_call(
        _tca_body,
        grid=(GRID,),
        in_specs=[_half_spec(), _row_spec(F), _full_spec((N, 1)),
                  _full_spec((F, H)), _full_spec((H,)), _full_spec((H, H))],
        out_specs=_half_spec(),
        out_shape=jax.ShapeDtypeStruct((2, N, 128), jnp.float32),
    )(s1p, xp, dinv, w1, b1, w2)


def _tcb(scat, pcat, dinv, b, w):
    return pl.pallas_call(
        _tcb_body,
        grid=(GRID,),
        in_specs=[_half_spec(), _half_spec(), _full_spec((N, 1)),
                  _full_spec((H,)), _full_spec((H, H))],
        out_specs=_half_spec(),
        out_shape=jax.ShapeDtypeStruct((2, N, 128), jnp.float32),
    )(scat, pcat, dinv, b, w)


def _tcc(scat, pcat, dinv, b, w4):
    return pl.pallas_call(
        _tcc_body,
        grid=(GRID,),
        in_specs=[_half_spec(), _half_spec(), _full_spec((N, 1)),
                  _full_spec((H,)), _full_spec((H, 1))],
        out_specs=_row_spec(1),
        out_shape=jax.ShapeDtypeStruct((N, 1), jnp.float32),
    )(scat, pcat, dinv, b, w4)


def _tcd(s4p, p4p, dinv, b4):
    return pl.pallas_call(
        _tcd_body,
        grid=(1,),
        in_specs=[_full_spec((_NTILE, N)), _full_spec((N, 1)),
                  _full_spec((N, 1)), _full_spec((1,))],
        out_specs=_full_spec((N, 1)),
        out_shape=jax.ShapeDtypeStruct((N, 1), jnp.float32),
    )(s4p, p4p, dinv, b4)


from jax.experimental.pallas import tpu_sc as plsc

_MESH = plsc.VectorSubcoreMesh(core_axis_name="c", subcore_axis_name="s")
_NTILE = 32
_RPT = 640
_WB = 80
_NQ = 8


def _zero16():
    return jnp.zeros((16,), jnp.float32)


def _splat(ref, e_idx):
    return plsc.load_gather(ref, [jnp.full((16,), e_idx, jnp.int32)])


def _deg_body(dst_hbm, ew_hbm, out_hbm, dstb, ewb, acc):
    c = lax.axis_index("c")
    s = lax.axis_index("s")
    ept = E // _NTILE
    wid = c * 16 + s
    base = wid * ept

    def zero(i, _):
        acc[pl.ds(i * 16, 16)] = _zero16()
        return 0
    lax.fori_loop(0, N // 16, zero, 0)

    pltpu.sync_copy(dst_hbm.at[pl.ds(base, ept)], dstb)
    pltpu.sync_copy(ew_hbm.at[pl.ds(base, ept)], ewb)

    def body(g, _):
        d16 = dstb[pl.ds(g * 16, 16)]
        w16 = ewb[pl.ds(g * 16, 16)]
        plsc.addupdate_scatter(acc, [d16], w16)
        return 0
    lax.fori_loop(0, ept // 16, body, 0)

    pltpu.sync_copy(acc, out_hbm.at[wid])


def _deg_partials(dst, ew):
    ept = E // _NTILE
    f = pl.kernel(
        _deg_body,
        out_type=jax.ShapeDtypeStruct((_NTILE, N), jnp.float32),
        mesh=_MESH,
        compiler_params=pltpu.CompilerParams(needs_layout_passes=False),
        scratch_types=[
            pltpu.VMEM((ept,), jnp.int32),
            pltpu.VMEM((ept,), jnp.float32),
            pltpu.VMEM((N,), jnp.float32),
        ],
    )
    return f(dst, ew)


def _scalar_body(p4_hbm, src_hbm, dst_hbm, ew_hbm, out_hbm,
                 pv, srcb, dstb, ewb, acc):
    c = lax.axis_index("c")
    s = lax.axis_index("s")
    ept = E // _NTILE
    wid = c * 16 + s
    base = wid * ept

    def zero(i, _):
        acc[pl.ds(i * 16, 16)] = _zero16()
        return 0
    lax.fori_loop(0, N // 16, zero, 0)

    pltpu.sync_copy(p4_hbm, pv)
    pltpu.sync_copy(src_hbm.at[pl.ds(base, ept)], srcb)
    pltpu.sync_copy(dst_hbm.at[pl.ds(base, ept)], dstb)
    pltpu.sync_copy(ew_hbm.at[pl.ds(base, ept)], ewb)

    def body(g, _):
        s16 = srcb[pl.ds(g * 16, 16)]
        d16 = dstb[pl.ds(g * 16, 16)]
        w16 = ewb[pl.ds(g * 16, 16)]
        vals = plsc.load_gather(pv, [s16]) * w16
        plsc.addupdate_scatter(acc, [d16], vals)
        return 0
    lax.fori_loop(0, ept // 16, body, 0)

    pltpu.sync_copy(acc, out_hbm.at[wid])


def _agg_scalar(p4, src, dst, ew):
    ept = E // _NTILE
    f = pl.kernel(
        _scalar_body,
        out_type=jax.ShapeDtypeStruct((_NTILE, N), jnp.float32),
        mesh=_MESH,
        compiler_params=pltpu.CompilerParams(needs_layout_passes=False),
        scratch_types=[
            pltpu.VMEM((N,), jnp.float32),
            pltpu.VMEM((ept,), jnp.int32),
            pltpu.VMEM((ept,), jnp.int32),
            pltpu.VMEM((ept,), jnp.float32),
            pltpu.VMEM((N,), jnp.float32),
        ],
    )
    return f(p4, src, dst, ew)


_K = 80


_EB = 2000


def _bcast(v16, jj):
    idx = jnp.full((16, 1), jj, jnp.int32)
    dn = lax.GatherDimensionNumbers(
        offset_dims=(), collapsed_slice_dims=(0,), start_index_map=(0,))
    return lax.gather(v16, idx, dn, (1,),
                      mode=lax.GatherScatterMode.PROMISE_IN_BOUNDS)


def _make_agg_body(col_split):
    ept = (E // 16) if col_split else (E // _NTILE)
    nblk = ept // _EB
    nch = _EB // _K

    def body(h_hbm, src_hbm, dst_hbm, ew_hbm, out_hbm,
             src_all, dst_all, ew_all, srcv2, dstv2, rows2, stage, sacc,
             sem_a, sem_b, sem_c, sem_sa, sem_sb, sem_sc):
        c = lax.axis_index("c")
        s = lax.axis_index("s")
        if col_split:
            ebase = s * ept
            roff = c * N
        else:
            ebase = (c * 16 + s) * ept
            roff = c * 0
        row0 = pl.multiple_of(s * _RPT, 8)
        nq = jnp.where(s == 15, 5, _NQ)

        def zrow(i, _):
            for u in range(8):
                stage[i, pl.ds(u * 16, 16)] = _zero16()
            return 0
        lax.fori_loop(0, _WB, zrow, 0)
        for q in range(_NQ):
            @pl.when(q < nq)
            def _():
                pltpu.sync_copy(stage, sacc.at[pl.ds(row0 + q * _WB, _WB)])
        plsc.subcore_barrier()

        sems = (sem_a, sem_b, sem_c)

        def build_idx(b, eb):
            for g in range(_K // 16):
                sl16 = pl.ds(eb + g * 16, 16)
                srcv2[b, pl.ds(g * 16, 16)] = src_all[sl16] + roff
                dstv2[b, pl.ds(g * 16, 16)] = dst_all[sl16]

        ssems = (sem_sa, sem_sb, sem_sc)

        def start_gather(b):
            pltpu.async_copy(h_hbm.at[srcv2.at[b]], rows2.at[b], sems[b])

        def wait_gather(b):
            pltpu.make_async_copy(h_hbm.at[srcv2.at[b]], rows2.at[b],
                                  sems[b]).wait()

        def wait_scatter(b):
            pltpu.make_async_copy(rows2.at[b], sacc.at[dstv2.at[b]],
                                  ssems[b]).wait()

        def process(t, b):
            bn = (b + 2) % 3
            @pl.when(t + 2 < nch)
            def _():
                @pl.when(t > 0)
                def _():
                    wait_scatter(bn)
                build_idx(bn, (t + 2) * _K)
                start_gather(bn)
            wait_gather(b)

            for g in range(_K // 16):
                w16 = ew_all[pl.ds(t * _K + g * 16, 16)]
                for jj in range(16):
                    spl = _bcast(w16, jj)
                    r = g * 16 + jj
                    for u in range(8):
                        sl = pl.ds(u * 16, 16)
                        rows2[b, r, sl] = rows2[b, r, sl] * spl
            pltpu.async_copy(rows2.at[b], sacc.at[dstv2.at[b]], ssems[b],
                             add=True)

        def blk(bi, _):
            bb = ebase + bi * _EB
            pltpu.sync_copy(src_hbm.at[pl.ds(bb, _EB)], src_all)
            pltpu.sync_copy(dst_hbm.at[pl.ds(bb, _EB)], dst_all)
            pltpu.sync_copy(ew_hbm.at[pl.ds(bb, _EB)], ew_all)
            build_idx(0, 0)
            start_gather(0)
            build_idx(1, _K)
            start_gather(1)

            def chunk(t, _):
                m = lax.rem(t, 3)
                @pl.when(m == 0)
                def _():
                    process(t, 0)

                @pl.when(m == 1)
                def _():
                    process(t, 1)

                @pl.when(m == 2)
                def _():
                    process(t, 2)
                return 0
            lax.fori_loop(0, nch, chunk, 0)
            wait_scatter(0)
            wait_scatter(1)
            wait_scatter(2)
            return 0
        lax.fori_loop(0, nblk, blk, 0)
        plsc.subcore_barrier()

        for q in range(_NQ):
            @pl.when(q < nq)
            def _():
                pltpu.sync_copy(sacc.at[pl.ds(row0 + q * _WB, _WB)], stage)
                pltpu.sync_copy(stage, out_hbm.at[c, pl.ds(row0 + q * _WB, _WB)])

    return body, ept


def _make_agg(col_split):
    body, ept = _make_agg_body(col_split)
    del ept
    return pl.kernel(
        body,
        out_type=jax.ShapeDtypeStruct((2, N, 128), jnp.float32),
        mesh=_MESH,
        compiler_params=pltpu.CompilerParams(needs_layout_passes=False),
        scratch_types=[
            pltpu.VMEM((_EB,), jnp.int32),
            pltpu.VMEM((_EB,), jnp.int32),
            pltpu.VMEM((_EB,), jnp.float32),
            pltpu.VMEM((3, _K), jnp.int32),
            pltpu.VMEM((3, _K), jnp.int32),
            pltpu.VMEM((3, _K, 128), jnp.float32),
            pltpu.VMEM((_WB, 128), jnp.float32),
            pltpu.VMEM_SHARED((N, 128), jnp.float32),
            pltpu.SemaphoreType.DMA,
            pltpu.SemaphoreType.DMA,
            pltpu.SemaphoreType.DMA,
            pltpu.SemaphoreType.DMA,
            pltpu.SemaphoreType.DMA,
            pltpu.SemaphoreType.DMA,
        ],
    )


_agg_cols_k = _make_agg(True)
_agg_edges_k = _make_agg(False)


def _agg_cols(hcat, src, dst, ew):
    return _agg_cols_k(hcat, src, dst, ew)


def _agg_edges(xp, src, dst, ew):
    return _agg_edges_k(xp, src, dst, ew)


def kernel(x, edge_index, edge_weight, W1, b1, W2, b2, W3, b3, W4, b4):
    src = edge_index[0]
    dst = edge_index[1]
    ew = edge_weight

    degp = _deg_partials(dst, ew)
    dinv, xp = _tc0(x, degp)

    s1p = _agg_edges(xp, src, dst, ew)
    p2cat = _tca(s1p, xp, dinv, W1, b1, W2)

    s2cat = _agg_cols(p2cat.reshape(2 * N, 128), src, dst, ew)
    p3cat = _tcb(s2cat, p2cat, dinv, b2, W3)

    s3cat = _agg_cols(p3cat.reshape(2 * N, 128), src, dst, ew)
    p4p = _tcc(s3cat, p3cat, dinv, b3, W4)

    s4p = _agg_scalar(p4p[:, 0], src, dst, ew)
    out = _tcd(s4p, p4p, dinv, b4)
    return out


# speedup vs baseline: 1.6205x; 1.6205x over previous
import functools

import jax
import jax.numpy as jnp
from jax import lax
from jax.experimental import pallas as pl
from jax.experimental.pallas import tpu as pltpu

N = 10000
E = 320000
F = 128
H = 256
RB = 2000
GRID = N // RB


def _gelu(v):
    return 0.5 * v * (1.0 + lax.erf(v * (2.0 ** -0.5)))


def _dinv_block(dinv_ref):
    i = pl.program_id(0)
    return dinv_ref[pl.ds(i * RB, RB), :]


def _tc0_body(x_ref, degp_ref, dinv_ref, xp_ref):
    deg = jnp.sum(degp_ref[...], axis=0) + 1.0
    dinv_ref[...] = lax.rsqrt(deg)[:, None]
    xp_ref[...] = x_ref[...] * _dinv_block(dinv_ref)


def _tca_body(s1p_ref, xp_ref, dinv_ref, w1_ref, b1_ref, w2_ref, p2_ref):
    dinv = _dinv_block(dinv_ref)
    g = dinv * (s1p_ref[0] + s1p_ref[1] + xp_ref[...])
    z1 = jnp.dot(g, w1_ref[...], preferred_element_type=jnp.float32) + b1_ref[...]
    a1 = _gelu(z1)
    p2 = jnp.dot(a1, w2_ref[...], preferred_element_type=jnp.float32) * dinv
    p2_ref[0] = p2[:, :128]
    p2_ref[1] = p2[:, 128:]


def _tcb_body(scat_ref, pcat_ref, dinv_ref, b_ref, w_ref, pn_ref):
    dinv = _dinv_block(dinv_ref)
    zlo = dinv * (scat_ref[0] + pcat_ref[0])
    zhi = dinv * (scat_ref[1] + pcat_ref[1])
    z = jnp.concatenate([zlo, zhi], axis=1) + b_ref[...]
    a = _gelu(z)
    pn = jnp.dot(a, w_ref[...], preferred_element_type=jnp.float32) * dinv
    pn_ref[0] = pn[:, :128]
    pn_ref[1] = pn[:, 128:]


def _tcc_body(scat_ref, pcat_ref, dinv_ref, b_ref, w4_ref, p4_ref):
    dinv = _dinv_block(dinv_ref)
    zlo = dinv * (scat_ref[0] + pcat_ref[0])
    zhi = dinv * (scat_ref[1] + pcat_ref[1])
    z = jnp.concatenate([zlo, zhi], axis=1) + b_ref[...]
    a = _gelu(z)
    p4_ref[...] = jnp.dot(a, w4_ref[...], preferred_element_type=jnp.float32) * dinv


def _tcd_body(s4p_ref, p4p_ref, dinv_ref, b4_ref, out_ref):
    s4 = jnp.sum(s4p_ref[...], axis=0)
    out_ref[...] = dinv_ref[...] * (s4[:, None] + p4p_ref[...]) + b4_ref[0]


def _row_spec(w):
    return pl.BlockSpec((RB, w), lambda i: (i, 0))


def _half_spec():
    return pl.BlockSpec((2, RB, 128), lambda i: (0, i, 0))


def _full_spec(shape):
    nd = len(shape)
    return pl.BlockSpec(shape, lambda i: (0,) * nd)


def _tc0(x, degp):
    return pl.pallas_call(
        _tc0_body,
        grid=(GRID,),
        in_specs=[_row_spec(F), _full_spec((_NTILE, N))],
        out_specs=[_full_spec((N, 1)), _row_spec(F)],
        out_shape=[jax.ShapeDtypeStruct((N, 1), jnp.float32),
                   jax.ShapeDtypeStruct((N, F), jnp.float32)],
    )(x, degp)


def _tca(s1p, xp, dinv, w1, b1, w2):
    return pl.pallas_call(
        _tca_body,
        grid=(GRID,),
        in_specs=[_half_spec(), _row_spec(F), _full_spec((N, 1)),
                  _full_spec((F, H)), _full_spec((H,)), _full_spec((H, H))],
        out_specs=_half_spec(),
        out_shape=jax.ShapeDtypeStruct((2, N, 128), jnp.float32),
    )(s1p, xp, dinv, w1, b1, w2)


def _tcb(scat, pcat, dinv, b, w):
    return pl.pallas_call(
        _tcb_body,
        grid=(GRID,),
        in_specs=[_half_spec(), _half_spec(), _full_spec((N, 1)),
                  _full_spec((H,)), _full_spec((H, H))],
        out_specs=_half_spec(),
        out_shape=jax.ShapeDtypeStruct((2, N, 128), jnp.float32),
    )(scat, pcat, dinv, b, w)


def _tcc(scat, pcat, dinv, b, w4):
    return pl.pallas_call(
        _tcc_body,
        grid=(GRID,),
        in_specs=[_half_spec(), _half_spec(), _full_spec((N, 1)),
                  _full_spec((H,)), _full_spec((H, 1))],
        out_specs=_row_spec(1),
        out_shape=jax.ShapeDtypeStruct((N, 1), jnp.float32),
    )(scat, pcat, dinv, b, w4)


def _tcd(s4p, p4p, dinv, b4):
    return pl.pallas_call(
        _tcd_body,
        grid=(1,),
        in_specs=[_full_spec((_NTILE, N)), _full_spec((N, 1)),
                  _full_spec((N, 1)), _full_spec((1,))],
        out_specs=_full_spec((N, 1)),
        out_shape=jax.ShapeDtypeStruct((N, 1), jnp.float32),
    )(s4p, p4p, dinv, b4)


from jax.experimental.pallas import tpu_sc as plsc

_MESH = plsc.VectorSubcoreMesh(core_axis_name="c", subcore_axis_name="s")
_NTILE = 32
_RPT = 640
_WB = 80
_NQ = 8


def _zero16():
    return jnp.zeros((16,), jnp.float32)


def _splat(ref, e_idx):
    return plsc.load_gather(ref, [jnp.full((16,), e_idx, jnp.int32)])


def _deg_body(dst_hbm, ew_hbm, out_hbm, dstb, ewb, acc):
    c = lax.axis_index("c")
    s = lax.axis_index("s")
    ept = E // _NTILE
    wid = c * 16 + s
    base = wid * ept

    def zero(i, _):
        acc[pl.ds(i * 16, 16)] = _zero16()
        return 0
    lax.fori_loop(0, N // 16, zero, 0)

    pltpu.sync_copy(dst_hbm.at[pl.ds(base, ept)], dstb)
    pltpu.sync_copy(ew_hbm.at[pl.ds(base, ept)], ewb)

    def body(g, _):
        d16 = dstb[pl.ds(g * 16, 16)]
        w16 = ewb[pl.ds(g * 16, 16)]
        plsc.addupdate_scatter(acc, [d16], w16)
        return 0
    lax.fori_loop(0, ept // 16, body, 0)

    pltpu.sync_copy(acc, out_hbm.at[wid])


def _deg_partials(dst, ew):
    ept = E // _NTILE
    f = pl.kernel(
        _deg_body,
        out_type=jax.ShapeDtypeStruct((_NTILE, N), jnp.float32),
        mesh=_MESH,
        compiler_params=pltpu.CompilerParams(needs_layout_passes=False),
        scratch_types=[
            pltpu.VMEM((ept,), jnp.int32),
            pltpu.VMEM((ept,), jnp.float32),
            pltpu.VMEM((N,), jnp.float32),
        ],
    )
    return f(dst, ew)


def _scalar_body(p4_hbm, src_hbm, dst_hbm, ew_hbm, out_hbm,
                 pv, srcb, dstb, ewb, acc):
    c = lax.axis_index("c")
    s = lax.axis_index("s")
    ept = E // _NTILE
    wid = c * 16 + s
    base = wid * ept

    def zero(i, _):
        acc[pl.ds(i * 16, 16)] = _zero16()
        return 0
    lax.fori_loop(0, N // 16, zero, 0)

    pltpu.sync_copy(p4_hbm, pv)
    pltpu.sync_copy(src_hbm.at[pl.ds(base, ept)], srcb)
    pltpu.sync_copy(dst_hbm.at[pl.ds(base, ept)], dstb)
    pltpu.sync_copy(ew_hbm.at[pl.ds(base, ept)], ewb)

    def body(g, _):
        s16 = srcb[pl.ds(g * 16, 16)]
        d16 = dstb[pl.ds(g * 16, 16)]
        w16 = ewb[pl.ds(g * 16, 16)]
        vals = plsc.load_gather(pv, [s16]) * w16
        plsc.addupdate_scatter(acc, [d16], vals)
        return 0
    lax.fori_loop(0, ept // 16, body, 0)

    pltpu.sync_copy(acc, out_hbm.at[wid])


def _agg_scalar(p4, src, dst, ew):
    ept = E // _NTILE
    f = pl.kernel(
        _scalar_body,
        out_type=jax.ShapeDtypeStruct((_NTILE, N), jnp.float32),
        mesh=_MESH,
        compiler_params=pltpu.CompilerParams(needs_layout_passes=False),
        scratch_types=[
            pltpu.VMEM((N,), jnp.float32),
            pltpu.VMEM((ept,), jnp.int32),
            pltpu.VMEM((ept,), jnp.int32),
            pltpu.VMEM((ept,), jnp.float32),
            pltpu.VMEM((N,), jnp.float32),
        ],
    )
    return f(p4, src, dst, ew)


_K = 80


_EB = 2000


def _bcast(v16, jj):
    idx = jnp.full((16, 1), jj, jnp.int32)
    dn = lax.GatherDimensionNumbers(
        offset_dims=(), collapsed_slice_dims=(0,), start_index_map=(0,))
    return lax.gather(v16, idx, dn, (1,),
                      mode=lax.GatherScatterMode.PROMISE_IN_BOUNDS)


def _make_agg_body(col_split):
    ept = (E // 16) if col_split else (E // _NTILE)
    eb_sz = 4000 if col_split else 2000
    nblk = ept // eb_sz
    nch = eb_sz // _K

    def body(h_hbm, src_hbm, dst_hbm, ew_hbm, out_hbm,
             src_all, dst_all, ew_all, srcv2, dstv2, rows2, stage, sacc,
             sem_a, sem_b, sem_sa, sem_sb):
        c = lax.axis_index("c")
        s = lax.axis_index("s")
        if col_split:
            ebase = s * ept
            roff = c * N
        else:
            ebase = (c * 16 + s) * ept
            roff = c * 0
        row0 = pl.multiple_of(s * _RPT, 8)
        nq = jnp.where(s == 15, 5, _NQ)

        def zrow(i, _):
            for u in range(8):
                stage[i, pl.ds(u * 16, 16)] = _zero16()
            return 0
        lax.fori_loop(0, _WB, zrow, 0)
        for q in range(_NQ):
            @pl.when(q < nq)
            def _():
                pltpu.sync_copy(stage, sacc.at[pl.ds(row0 + q * _WB, _WB)])
        plsc.subcore_barrier()

        sems = (sem_a, sem_b)

        def build_idx(b, eb):
            for g in range(_K // 16):
                sl16 = pl.ds(eb + g * 16, 16)
                srcv2[b, pl.ds(g * 16, 16)] = src_all[sl16] + roff
                dstv2[b, pl.ds(g * 16, 16)] = dst_all[sl16]

        ssems = (sem_sa, sem_sb)

        def start_gather(b):
            pltpu.async_copy(h_hbm.at[srcv2.at[b]], rows2.at[b], sems[b])

        def wait_gather(b):
            pltpu.make_async_copy(h_hbm.at[srcv2.at[b]], rows2.at[b],
                                  sems[b]).wait()

        def wait_scatter(b):
            pltpu.make_async_copy(rows2.at[b], sacc.at[dstv2.at[b]],
                                  ssems[b]).wait()

        def process(t, b):
            @pl.when(t + 1 < nch)
            def _():
                @pl.when(t > 0)
                def _():
                    wait_scatter(1 - b)
                build_idx(1 - b, (t + 1) * _K)
                start_gather(1 - b)
            wait_gather(b)

            for g in range(_K // 16):
                w16 = ew_all[pl.ds(t * _K + g * 16, 16)]
                for jj in range(16):
                    spl = _bcast(w16, jj)
                    r = g * 16 + jj
                    for u in range(8):
                        sl = pl.ds(u * 16, 16)
                        rows2[b, r, sl] = rows2[b, r, sl] * spl
            pltpu.async_copy(rows2.at[b], sacc.at[dstv2.at[b]], ssems[b],
                             add=True)

        def blk(bi, _):
            bb = ebase + bi * eb_sz
            pltpu.sync_copy(src_hbm.at[pl.ds(bb, eb_sz)], src_all)
            pltpu.sync_copy(dst_hbm.at[pl.ds(bb, eb_sz)], dst_all)
            pltpu.sync_copy(ew_hbm.at[pl.ds(bb, eb_sz)], ew_all)
            build_idx(0, 0)
            start_gather(0)

            def chunk(t, _):
                @pl.when(t % 2 == 0)
                def _():
                    process(t, 0)

                @pl.when(t % 2 == 1)
                def _():
                    process(t, 1)
                return 0
            lax.fori_loop(0, nch, chunk, 0)
            wait_scatter(0)
            wait_scatter(1)
            return 0
        lax.fori_loop(0, nblk, blk, 0)
        plsc.subcore_barrier()

        for q in range(_NQ):
            @pl.when(q < nq)
            def _():
                pltpu.sync_copy(sacc.at[pl.ds(row0 + q * _WB, _WB)], stage)
                pltpu.sync_copy(stage, out_hbm.at[c, pl.ds(row0 + q * _WB, _WB)])

    return body, ept


def _make_agg(col_split):
    body, ept = _make_agg_body(col_split)
    del ept
    eb_sz = 4000 if col_split else 2000
    return pl.kernel(
        body,
        out_type=jax.ShapeDtypeStruct((2, N, 128), jnp.float32),
        mesh=_MESH,
        compiler_params=pltpu.CompilerParams(needs_layout_passes=False),
        scratch_types=[
            pltpu.VMEM((eb_sz,), jnp.int32),
            pltpu.VMEM((eb_sz,), jnp.int32),
            pltpu.VMEM((eb_sz,), jnp.float32),
            pltpu.VMEM((2, _K), jnp.int32),
            pltpu.VMEM((2, _K), jnp.int32),
            pltpu.VMEM((2, _K, 128), jnp.float32),
            pltpu.VMEM((_WB, 128), jnp.float32),
            pltpu.VMEM_SHARED((N, 128), jnp.float32),
            pltpu.SemaphoreType.DMA,
            pltpu.SemaphoreType.DMA,
            pltpu.SemaphoreType.DMA,
            pltpu.SemaphoreType.DMA,
        ],
    )


_agg_cols_k = _make_agg(True)
_agg_edges_k = _make_agg(False)


def _agg_cols(hcat, src, dst, ew):
    return _agg_cols_k(hcat, src, dst, ew)


def _agg_edges(xp, src, dst, ew):
    return _agg_edges_k(xp, src, dst, ew)


def kernel(x, edge_index, edge_weight, W1, b1, W2, b2, W3, b3, W4, b4):
    src = edge_index[0]
    dst = edge_index[1]
    ew = edge_weight

    degp = _deg_partials(dst, ew)
    dinv, xp = _tc0(x, degp)

    s1p = _agg_edges(xp, src, dst, ew)
    p2cat = _tca(s1p, xp, dinv, W1, b1, W2)

    s2cat = _agg_cols(p2cat.reshape(2 * N, 128), src, dst, ew)
    p3cat = _tcb(s2cat, p2cat, dinv, b2, W3)

    s3cat = _agg_cols(p3cat.reshape(2 * N, 128), src, dst, ew)
    p4p = _tcc(s3cat, p3cat, dinv, b3, W4)

    s4p = _agg_scalar(p4p[:, 0], src, dst, ew)
    out = _tcd(s4p, p4p, dinv, b4)
    return out
